# Initial kernel scaffold; baseline (speedup 1.0000x reference)
import jax
import jax.numpy as jnp
from jax import lax
from jax.experimental import pallas as pl
from jax.experimental.pallas import tpu as pltpu
from jax.experimental.pallas import tpu_sc as plsc

B = 4096
L = 50
ROWS = 2 * B
WD = 32
ED = 64
NW = 32
RPW = ROWS // NW
IPW = RPW * L
CHUNK = 128
NCHUNK = IPW // CHUNK


def _pool_body(wt_hbm, et_hbm, widx_hbm, eidx_hbm, gidx_hbm, z32_hbm, z64_hbm,
               outw_hbm, oute_hbm,
               idxw_v, idxe_v, g_v, roww_v, rowe_v, accw_s, acce_s, sem):
    c = lax.axis_index("c")
    s = lax.axis_index("s")
    w = c * 16 + s

    pltpu.sync_copy(widx_hbm.at[pl.ds(w * NCHUNK, NCHUNK)], idxw_v)
    pltpu.sync_copy(eidx_hbm.at[pl.ds(w * NCHUNK, NCHUNK)], idxe_v)
    pltpu.sync_copy(gidx_hbm.at[s], g_v)

    pltpu.sync_copy(z32_hbm, accw_s.at[pl.ds(s * RPW, RPW)])
    pltpu.sync_copy(z64_hbm, acce_s.at[pl.ds(s * RPW, RPW)])

    def word_step(i, carry):
        pltpu.async_copy(wt_hbm.at[idxw_v.at[i]], roww_v, sem).wait()
        pltpu.sync_copy(roww_v, accw_s.at[g_v.at[i]], add=True)
        return carry

    lax.fori_loop(0, NCHUNK, word_step, 0, unroll=False)

    def ent_step(i, carry):
        pltpu.async_copy(et_hbm.at[idxe_v.at[i]], rowe_v, sem).wait()
        pltpu.sync_copy(rowe_v, acce_s.at[g_v.at[i]], add=True)
        return carry

    lax.fori_loop(0, NCHUNK, ent_step, 0, unroll=False)

    pltpu.sync_copy(accw_s.at[pl.ds(s * RPW, RPW)],
                    outw_hbm.at[pl.ds(w * RPW, RPW)])
    pltpu.sync_copy(acce_s.at[pl.ds(s * RPW, RPW)],
                    oute_hbm.at[pl.ds(w * RPW, RPW)])


def _scorer_body(params_ref, wsum_ref, esum_ref, erw_ref, elw_ref,
                 ecww_ref, ecwe_ref, out_ref):
    wpool = wsum_ref[...] * (1.0 / L)
    epool = esum_ref[...] * (1.0 / L)
    er_raw = jnp.sum(wpool * erw_ref[...], axis=1, keepdims=True)
    el_raw = jnp.sum(epool * elw_ref[...], axis=1, keepdims=True)
    ec_raw = (jnp.sum(wpool * ecww_ref[...], axis=1, keepdims=True)
              + jnp.sum(epool * ecwe_ref[...], axis=1, keepdims=True))
    er_b, el_b, ec_b, cl_b = (params_ref[0], params_ref[1], params_ref[2],
                              params_ref[3])
    cl0, cl1, cl2 = params_ref[4], params_ref[5], params_ref[6]
    er = jax.nn.relu(er_raw + er_b - 0.5) + 0.5
    el = jax.nn.relu(el_raw + el_b - 0.5) + 0.5
    ec = jax.nn.sigmoid(ec_raw + ec_b)
    out_ref[...] = jax.nn.sigmoid(er * cl0 + el * cl1 + ec * cl2 + cl_b)


def kernel(lctx_words, rctx_words, lctx_entities, rctx_entities,
           word_table, entity_table, er_w, er_b, el_w, el_b,
           ec_w, ec_b, cl_w, cl_b):
    widx = jnp.concatenate([rctx_words, lctx_words], axis=0).reshape(
        NW * NCHUNK, CHUNK)
    eidx = jnp.concatenate([rctx_entities, lctx_entities], axis=0).reshape(
        NW * NCHUNK, CHUNK)
    seg = (jnp.arange(IPW, dtype=jnp.int32) // L).reshape(1, NCHUNK, CHUNK)
    gidx = seg + (RPW * jnp.arange(16, dtype=jnp.int32))[:, None, None]
    z32 = jnp.zeros((RPW, WD), jnp.float32)
    z64 = jnp.zeros((RPW, ED), jnp.float32)

    mesh = plsc.VectorSubcoreMesh(core_axis_name="c", subcore_axis_name="s")
    pool = pl.kernel(
        _pool_body,
        out_type=(jax.ShapeDtypeStruct((ROWS, WD), jnp.float32),
                  jax.ShapeDtypeStruct((ROWS, ED), jnp.float32)),
        mesh=mesh,
        scratch_types=[
            pltpu.VMEM((NCHUNK, CHUNK), jnp.int32),
            pltpu.VMEM((NCHUNK, CHUNK), jnp.int32),
            pltpu.VMEM((NCHUNK, CHUNK), jnp.int32),
            pltpu.VMEM((CHUNK, WD), jnp.float32),
            pltpu.VMEM((CHUNK, ED), jnp.float32),
            pltpu.VMEM_SHARED((16 * RPW, WD), jnp.float32),
            pltpu.VMEM_SHARED((16 * RPW, ED), jnp.float32),
            pltpu.SemaphoreType.DMA,
        ],
    )
    wsum, esum = pool(word_table, entity_table, widx, eidx, gidx, z32, z64)

    params = jnp.concatenate([er_b, el_b, ec_b, cl_b, cl_w[:, 0]])
    final = pl.pallas_call(
        _scorer_body,
        out_shape=jax.ShapeDtypeStruct((ROWS, 1), jnp.float32),
        in_specs=[
            pl.BlockSpec(memory_space=pltpu.SMEM),
            pl.BlockSpec(memory_space=pltpu.VMEM),
            pl.BlockSpec(memory_space=pltpu.VMEM),
            pl.BlockSpec(memory_space=pltpu.VMEM),
            pl.BlockSpec(memory_space=pltpu.VMEM),
            pl.BlockSpec(memory_space=pltpu.VMEM),
            pl.BlockSpec(memory_space=pltpu.VMEM),
        ],
    )(params, wsum, esum, er_w.reshape(1, WD), el_w.reshape(1, ED),
      ec_w[:WD].reshape(1, WD), ec_w[WD:].reshape(1, ED))
    return final


# speedup vs baseline: 3.4118x; 3.4118x over previous
import jax
import jax.numpy as jnp
from jax import lax
from jax.experimental import pallas as pl
from jax.experimental.pallas import tpu as pltpu
from jax.experimental.pallas import tpu_sc as plsc

B = 4096
L = 50
ROWS = 2 * B
WD = 32
ED = 64
NW = 32
RPW = ROWS // NW
IPW = RPW * L
CHUNK = 128
NCHUNK = IPW // CHUNK


def _pool_body(wt_hbm, et_hbm, widx_hbm, eidx_hbm, gidx_hbm, z32_hbm, z64_hbm,
               outw_hbm, oute_hbm,
               idxw_v, idxe_v, g_v, roww_v, rowe_v, accw_s, acce_s, sem):
    c = lax.axis_index("c")
    s = lax.axis_index("s")
    w = c * 16 + s

    pltpu.sync_copy(widx_hbm.at[w], idxw_v)
    pltpu.sync_copy(eidx_hbm.at[w], idxe_v)
    pltpu.sync_copy(gidx_hbm.at[s], g_v)

    pltpu.sync_copy(z32_hbm, accw_s.at[pl.ds(s * RPW, RPW)])
    pltpu.sync_copy(z64_hbm, acce_s.at[pl.ds(s * RPW, RPW)])

    def word_step(i, carry):
        pltpu.async_copy(wt_hbm.at[idxw_v.at[i]], roww_v, sem).wait()
        pltpu.sync_copy(roww_v, accw_s.at[g_v.at[i]], add=True)
        return carry

    lax.fori_loop(0, NCHUNK, word_step, 0, unroll=False)

    def ent_step(i, carry):
        pltpu.async_copy(et_hbm.at[idxe_v.at[i]], rowe_v, sem).wait()
        pltpu.sync_copy(rowe_v, acce_s.at[g_v.at[i]], add=True)
        return carry

    lax.fori_loop(0, NCHUNK, ent_step, 0, unroll=False)

    pltpu.sync_copy(accw_s.at[pl.ds(s * RPW, RPW)],
                    outw_hbm.at[pl.ds(w * RPW, RPW)])
    pltpu.sync_copy(acce_s.at[pl.ds(s * RPW, RPW)],
                    oute_hbm.at[pl.ds(w * RPW, RPW)])


def _scorer_body(params_ref, wsum_ref, esum_ref, erw_ref, elw_ref,
                 ecww_ref, ecwe_ref, out_ref):
    wpool = wsum_ref[...] * (1.0 / L)
    epool = esum_ref[...] * (1.0 / L)
    er_raw = jnp.sum(wpool * erw_ref[...], axis=1, keepdims=True)
    el_raw = jnp.sum(epool * elw_ref[...], axis=1, keepdims=True)
    ec_raw = (jnp.sum(wpool * ecww_ref[...], axis=1, keepdims=True)
              + jnp.sum(epool * ecwe_ref[...], axis=1, keepdims=True))
    er_b, el_b, ec_b, cl_b = (params_ref[0], params_ref[1], params_ref[2],
                              params_ref[3])
    cl0, cl1, cl2 = params_ref[4], params_ref[5], params_ref[6]
    er = jax.nn.relu(er_raw + er_b - 0.5) + 0.5
    el = jax.nn.relu(el_raw + el_b - 0.5) + 0.5
    ec = jax.nn.sigmoid(ec_raw + ec_b)
    out_ref[...] = jax.nn.sigmoid(er * cl0 + el * cl1 + ec * cl2 + cl_b)


def kernel(lctx_words, rctx_words, lctx_entities, rctx_entities,
           word_table, entity_table, er_w, er_b, el_w, el_b,
           ec_w, ec_b, cl_w, cl_b):
    widx = jnp.concatenate([rctx_words, lctx_words], axis=0).reshape(
        NW, NCHUNK, CHUNK)
    eidx = jnp.concatenate([rctx_entities, lctx_entities], axis=0).reshape(
        NW, NCHUNK, CHUNK)
    seg = (jnp.arange(IPW, dtype=jnp.int32) // L).reshape(1, NCHUNK, CHUNK)
    gidx = seg + (RPW * jnp.arange(16, dtype=jnp.int32))[:, None, None]
    z32 = jnp.zeros((RPW, WD), jnp.float32)
    z64 = jnp.zeros((RPW, ED), jnp.float32)

    mesh = plsc.VectorSubcoreMesh(core_axis_name="c", subcore_axis_name="s")
    pool = pl.kernel(
        _pool_body,
        out_type=(jax.ShapeDtypeStruct((ROWS, WD), jnp.float32),
                  jax.ShapeDtypeStruct((ROWS, ED), jnp.float32)),
        mesh=mesh,
        compiler_params=pltpu.CompilerParams(use_tc_tiling_on_sc=False),
        scratch_types=[
            pltpu.VMEM((NCHUNK, CHUNK), jnp.int32),
            pltpu.VMEM((NCHUNK, CHUNK), jnp.int32),
            pltpu.VMEM((NCHUNK, CHUNK), jnp.int32),
            pltpu.VMEM((CHUNK, WD), jnp.float32),
            pltpu.VMEM((CHUNK, ED), jnp.float32),
            pltpu.VMEM_SHARED((16 * RPW, WD), jnp.float32),
            pltpu.VMEM_SHARED((16 * RPW, ED), jnp.float32),
            pltpu.SemaphoreType.DMA,
        ],
    )
    wsum, esum = pool(word_table, entity_table, widx, eidx, gidx, z32, z64)

    params = jnp.concatenate([er_b, el_b, ec_b, cl_b, cl_w[:, 0]])
    final = pl.pallas_call(
        _scorer_body,
        out_shape=jax.ShapeDtypeStruct((ROWS, 1), jnp.float32),
        in_specs=[
            pl.BlockSpec(memory_space=pltpu.SMEM),
            pl.BlockSpec(memory_space=pltpu.VMEM),
            pl.BlockSpec(memory_space=pltpu.VMEM),
            pl.BlockSpec(memory_space=pltpu.VMEM),
            pl.BlockSpec(memory_space=pltpu.VMEM),
            pl.BlockSpec(memory_space=pltpu.VMEM),
            pl.BlockSpec(memory_space=pltpu.VMEM),
        ],
    )(params, wsum, esum, er_w.reshape(1, WD), el_w.reshape(1, ED),
      ec_w[:WD].reshape(1, WD), ec_w[WD:].reshape(1, ED))
    return final
